# baseline (speedup 1.0000x reference)
import functools

import jax
import jax.numpy as jnp
import numpy as np
from jax import lax
from jax.experimental import pallas as pl
from jax.experimental.pallas import tpu as pltpu
from jax.experimental.pallas import tpu_sc as plsc

N = 10000
E = 320000
F_IN = 128
F_OUT = 128
T = 4
D = T * F_OUT

NC = 2
NS = 16
NW = NC * NS
EPW = E // NW
CHUNK = 40
NCHUNK = EPW // CHUNK
NPAD = 10240
ROWS_PER_SUB = NPAD // NS


def _matmul_y(x, kcat2):
    def body(x_ref, k_ref, o_ref):
        r = jnp.dot(x_ref[...], k_ref[...],
                    preferred_element_type=jnp.float32)
        ra = r[:, :D // 2].astype(jnp.bfloat16).astype(jnp.float32)
        rb = r[:, D // 2:].astype(jnp.bfloat16).astype(jnp.float32)
        abits = lax.shift_right_logical(
            lax.bitcast_convert_type(ra, jnp.int32), 16)
        bbits = lax.bitcast_convert_type(rb, jnp.int32) & jnp.int32(-65536)
        o_ref[...] = abits | bbits

    return pl.pallas_call(
        body,
        grid=(10,),
        in_specs=[
            pl.BlockSpec((N // 10, F_IN), lambda i: (i, 0)),
            pl.BlockSpec((F_IN, D), lambda i: (0, 0)),
        ],
        out_specs=pl.BlockSpec((N // 10, D // 2), lambda i: (i, 0)),
        out_shape=jax.ShapeDtypeStruct((N, D // 2), jnp.int32),
    )(x, kcat2)


def _epilogue(partials, bias_row):
    def body(p_ref, b_ref, o_ref):
        o_ref[...] = jnp.maximum(p_ref[0] + p_ref[1] + b_ref[...], 0.0)

    return pl.pallas_call(
        body,
        grid=(10,),
        in_specs=[
            pl.BlockSpec((2, N // 10, F_OUT), lambda i: (0, i, 0)),
            pl.BlockSpec((1, F_OUT), lambda i: (0, 0)),
        ],
        out_specs=pl.BlockSpec((N // 10, F_OUT), lambda i: (i, 0)),
        out_shape=jax.ShapeDtypeStruct((N, F_OUT), jnp.float32),
    )(partials, bias_row)


def _make_sc_kernel():
    mesh = plsc.VectorSubcoreMesh(core_axis_name="c", subcore_axis_name="s",
                                  num_cores=NC, num_subcores=NS)

    @functools.partial(
        pl.kernel,
        out_type=jax.ShapeDtypeStruct((NC, NPAD, F_OUT), jnp.float32),
        mesh=mesh,
        compiler_params=pltpu.CompilerParams(needs_layout_passes=False),
        scratch_types=[
            pltpu.VMEM((CHUNK,), jnp.int32),
            pltpu.VMEM((CHUNK,), jnp.int32),
            pltpu.VMEM((CHUNK,), jnp.int32),
            pltpu.VMEM((CHUNK,), jnp.int32),
            pltpu.VMEM((CHUNK,), jnp.int32),
            pltpu.VMEM((CHUNK,), jnp.int32),
            pltpu.VMEM((CHUNK,), jnp.int32),
            pltpu.VMEM((CHUNK,), jnp.int32),
            pltpu.VMEM((CHUNK * T + 16,), jnp.float32),
            pltpu.VMEM((CHUNK * T + 16,), jnp.float32),
            pltpu.VMEM((CHUNK * T + 16,), jnp.float32),
            pltpu.VMEM((CHUNK * T + 16,), jnp.float32),
            pltpu.VMEM((CHUNK, D // 2), jnp.int32),
            pltpu.VMEM((CHUNK, D // 2), jnp.int32),
            pltpu.VMEM((CHUNK, D // 2), jnp.int32),
            pltpu.VMEM((CHUNK, D // 2), jnp.int32),
            pltpu.VMEM((CHUNK, F_OUT), jnp.float32),
            pltpu.VMEM_SHARED((NPAD, F_OUT), jnp.float32),
            pltpu.SemaphoreType.DMA,
            pltpu.SemaphoreType.DMA,
            pltpu.SemaphoreType.DMA,
            pltpu.SemaphoreType.DMA,
            pltpu.SemaphoreType.DMA,
            pltpu.SemaphoreType.DMA,
            pltpu.SemaphoreType.DMA,
            pltpu.SemaphoreType.DMA,
        ],
    )
    def sc_kernel(y_hbm, cols_hbm, rows_hbm, w_hbm, out_hbm,
                  cidx0_v, cidx1_v, cidx2_v, cidx3_v,
                  ridx0_v, ridx1_v, ridx2_v, ridx3_v,
                  w0_v, w1_v, w2_v, w3_v,
                  g0_v, g1_v, g2_v, g3_v, o_v, acc_sh,
                  isem0, isem1, isem2, isem3,
                  gsem0, gsem1, gsem2, gsem3):
        cid = lax.axis_index("c")
        sid = lax.axis_index("s")
        cidx_b = (cidx0_v, cidx1_v, cidx2_v, cidx3_v)
        ridx_b = (ridx0_v, ridx1_v, ridx2_v, ridx3_v)
        w_b = (w0_v, w1_v, w2_v, w3_v)
        g_b = (g0_v, g1_v, g2_v, g3_v)
        isem = (isem0, isem1, isem2, isem3)
        gsem = (gsem0, gsem1, gsem2, gsem3)

        zero16 = jnp.zeros((16,), jnp.float32)

        def zinit(i, _):
            for j in range(F_OUT // 16):
                o_v[i, pl.ds(j * 16, 16)] = zero16
            return 0

        lax.fori_loop(0, CHUNK, zinit, 0)
        for zc in range(ROWS_PER_SUB // CHUNK):
            r0 = sid * ROWS_PER_SUB + zc * CHUNK
            pltpu.sync_copy(o_v, acc_sh.at[pl.ds(r0, CHUNK), :])
        plsc.subcore_barrier()

        wid = sid * NC + cid

        def idx_copies(i, b):
            base = wid * EPW + i * CHUNK
            return (
                pltpu.make_async_copy(cols_hbm.at[pl.ds(base, CHUNK)],
                                      cidx_b[b], isem[b]),
                pltpu.make_async_copy(rows_hbm.at[pl.ds(base, CHUNK)],
                                      ridx_b[b], isem[b]),
                pltpu.make_async_copy(
                    w_hbm.at[pl.ds(base * T, CHUNK * T)],
                    w_b[b].at[pl.ds(0, CHUNK * T)], isem[b]),
            )

        def issue_idx(i, b):
            for c in idx_copies(i, b):
                c.start()

        def wait_idx(i, b):
            for c in idx_copies(i, b):
                c.wait()

        def g_copy(b):
            return pltpu.make_async_copy(y_hbm.at[cidx_b[b]],
                                         g_b[b], gsem[b])

        def compute_scatter(b):
            g_v = g_b[b]

            himask = jnp.full((16,), -65536, jnp.int32)

            @plsc.parallel_loop(0, CHUNK, unroll=8)
            def edge_body(e):
                wv = w_b[b][pl.ds(e * T, 16)]
                ws = (wv[0], wv[1], wv[2], wv[3])
                for k in range(F_OUT // 32):
                    acc_lo = jnp.zeros((16,), jnp.float32)
                    acc_hi = jnp.zeros((16,), jnp.float32)
                    for t in range(T):
                        v = g_v[e, pl.ds(t * (F_OUT // 2) + k * 16, 16)]
                        lo = plsc.bitcast(lax.shift_left(v, 16), jnp.float32)
                        hi = plsc.bitcast(v & himask, jnp.float32)
                        acc_lo = acc_lo + ws[t] * lo
                        acc_hi = acc_hi + ws[t] * hi
                    o_v[e, pl.ds(k * 32, 16)] = acc_lo
                    o_v[e, pl.ds(k * 32 + 16, 16)] = acc_hi
            pltpu.sync_copy(o_v, acc_sh.at[ridx_b[b]], add=True)

        for j in range(4):
            issue_idx(j, j)
        for j in range(3):
            wait_idx(j, j)
            g_copy(j).start()

        def quad_body(iq, _):
            for r in range(4):
                i = iq * 4 + r
                g_copy(r).wait()
                wait_idx(i + 3, (r + 3) % 4)
                g_copy((r + 3) % 4).start()
                compute_scatter(r)
                issue_idx(i + 4, r)
            return 0

        NQ = (NCHUNK - 6) // 4
        lax.fori_loop(0, NQ, quad_body, 0)
        for i in range(NCHUNK - 6, NCHUNK):
            r = i % 4
            g_copy(r).wait()
            if i + 3 < NCHUNK:
                wait_idx(i + 3, (i + 3) % 4)
                g_copy((i + 3) % 4).start()
            compute_scatter(r)
            if i + 4 < NCHUNK:
                issue_idx(i + 4, r)

        plsc.subcore_barrier()
        r0 = sid * ROWS_PER_SUB
        pltpu.sync_copy(acc_sh.at[pl.ds(r0, ROWS_PER_SUB), :],
                        out_hbm.at[cid, pl.ds(r0, ROWS_PER_SUB), :])

    return sc_kernel


def kernel(node_features, edge_features, indices, out_size, kernel, bias):
    kcat = jnp.transpose(kernel, (1, 0, 2)).reshape(F_IN, D)
    m = np.arange(D // 2)
    perm_a = (m // 64) * 128 + 32 * ((m % 64) // 16) + (m % 16)
    kcat2 = jnp.concatenate([kcat[:, perm_a], kcat[:, perm_a + 16]], axis=1)
    cols = indices[:, 1]
    rows = indices[:, 0]
    w_em = edge_features.T.reshape(-1)
    bias_adj = bias + (jnp.asarray(out_size, jnp.float32) - float(N))

    y_i32 = _matmul_y(node_features, kcat2)
    partials = _make_sc_kernel()(y_i32, cols, rows, w_em)
    return _epilogue(partials, bias_adj.reshape(1, F_OUT))


# speedup vs baseline: 1.0974x; 1.0974x over previous
import functools

import jax
import jax.numpy as jnp
import numpy as np
from jax import lax
from jax.experimental import pallas as pl
from jax.experimental.pallas import tpu as pltpu
from jax.experimental.pallas import tpu_sc as plsc

N = 10000
E = 320000
F_IN = 128
F_OUT = 128
T = 4
D = T * F_OUT

NC = 2
NS = 16
NW = NC * NS
EPW = E // NW
CHUNK = 40
NCHUNK = EPW // CHUNK
NPAD = 10240
ROWS_PER_SUB = NPAD // NS


def _matmul_y(x, kcat2):
    def body(x_ref, k_ref, o_ref):
        r = jnp.dot(x_ref[...], k_ref[...],
                    preferred_element_type=jnp.float32)
        ra = r[:, :D // 2].astype(jnp.bfloat16).astype(jnp.float32)
        rb = r[:, D // 2:].astype(jnp.bfloat16).astype(jnp.float32)
        abits = lax.shift_right_logical(
            lax.bitcast_convert_type(ra, jnp.int32), 16)
        bbits = lax.bitcast_convert_type(rb, jnp.int32) & jnp.int32(-65536)
        o_ref[...] = abits | bbits

    return pl.pallas_call(
        body,
        grid=(10,),
        in_specs=[
            pl.BlockSpec((N // 10, F_IN), lambda i: (i, 0)),
            pl.BlockSpec((F_IN, D), lambda i: (0, 0)),
        ],
        out_specs=pl.BlockSpec((N // 10, D // 2), lambda i: (i, 0)),
        out_shape=jax.ShapeDtypeStruct((N, D // 2), jnp.int32),
    )(x, kcat2)


def _epilogue(partials, bias_row):
    def body(p_ref, b_ref, o_ref):
        o_ref[...] = jnp.maximum(p_ref[0] + p_ref[1] + b_ref[...], 0.0)

    return pl.pallas_call(
        body,
        grid=(10,),
        in_specs=[
            pl.BlockSpec((2, N // 10, F_OUT), lambda i: (0, i, 0)),
            pl.BlockSpec((1, F_OUT), lambda i: (0, 0)),
        ],
        out_specs=pl.BlockSpec((N // 10, F_OUT), lambda i: (i, 0)),
        out_shape=jax.ShapeDtypeStruct((N, F_OUT), jnp.float32),
    )(partials, bias_row)


def _make_sc_kernel():
    mesh = plsc.VectorSubcoreMesh(core_axis_name="c", subcore_axis_name="s",
                                  num_cores=NC, num_subcores=NS)

    @functools.partial(
        pl.kernel,
        out_type=jax.ShapeDtypeStruct((NC, NPAD, F_OUT), jnp.float32),
        mesh=mesh,
        compiler_params=pltpu.CompilerParams(needs_layout_passes=False),
        scratch_types=[
            pltpu.VMEM((CHUNK,), jnp.int32),
            pltpu.VMEM((CHUNK,), jnp.int32),
            pltpu.VMEM((CHUNK,), jnp.int32),
            pltpu.VMEM((CHUNK,), jnp.int32),
            pltpu.VMEM((CHUNK,), jnp.int32),
            pltpu.VMEM((CHUNK,), jnp.int32),
            pltpu.VMEM((CHUNK * T + 16,), jnp.float32),
            pltpu.VMEM((CHUNK * T + 16,), jnp.float32),
            pltpu.VMEM((CHUNK, D // 2), jnp.int32),
            pltpu.VMEM((CHUNK, D // 2), jnp.int32),
            pltpu.VMEM((CHUNK, F_OUT), jnp.float32),
            pltpu.VMEM((CHUNK, F_OUT), jnp.float32),
            pltpu.VMEM_SHARED((NPAD, F_OUT), jnp.float32),
            pltpu.SemaphoreType.DMA,
            pltpu.SemaphoreType.DMA,
            pltpu.SemaphoreType.DMA,
            pltpu.SemaphoreType.DMA,
            pltpu.SemaphoreType.DMA,
            pltpu.SemaphoreType.DMA,
            pltpu.SemaphoreType.DMA,
            pltpu.SemaphoreType.DMA,
            pltpu.SemaphoreType.DMA,
            pltpu.SemaphoreType.DMA,
        ],
    )
    def sc_kernel(y_hbm, cols_hbm, rows_hbm, w_hbm, out_hbm,
                  cidx0_v, cidx1_v,
                  sidx0_v, sidx1_v, sidx2_v, sidx3_v,
                  w0_v, w1_v, g0_v, g1_v, o0_v, o1_v, acc_sh,
                  isem0, isem1, gsem0, gsem1,
                  rsem0, rsem1, rsem2, rsem3, ssem0, ssem1):
        cid = lax.axis_index("c")
        sid = lax.axis_index("s")
        cidx_b = (cidx0_v, cidx1_v)
        sidx_b = (sidx0_v, sidx1_v, sidx2_v, sidx3_v)
        w_b = (w0_v, w1_v)
        g_b = (g0_v, g1_v)
        o_b = (o0_v, o1_v)
        isem = (isem0, isem1)
        gsem = (gsem0, gsem1)
        rsem = (rsem0, rsem1, rsem2, rsem3)
        ssem = (ssem0, ssem1)

        zero16 = jnp.zeros((16,), jnp.float32)

        def zinit(i, _):
            for j in range(F_OUT // 16):
                o0_v[i, pl.ds(j * 16, 16)] = zero16
            return 0

        lax.fori_loop(0, CHUNK, zinit, 0)
        for zc in range(ROWS_PER_SUB // CHUNK):
            r0 = sid * ROWS_PER_SUB + zc * CHUNK
            pltpu.sync_copy(o0_v, acc_sh.at[pl.ds(r0, CHUNK), :])
        plsc.subcore_barrier()

        wid = sid * NC + cid

        def idx_copies(i, b):
            base = wid * EPW + i * CHUNK
            return (
                pltpu.make_async_copy(cols_hbm.at[pl.ds(base, CHUNK)],
                                      cidx_b[b], isem[b]),
                pltpu.make_async_copy(
                    w_hbm.at[pl.ds(base * T, CHUNK * T)],
                    w_b[b].at[pl.ds(0, CHUNK * T)], isem[b]),
            )

        def issue_idx(i, b):
            for c in idx_copies(i, b):
                c.start()

        def wait_idx(i, b):
            for c in idx_copies(i, b):
                c.wait()

        def rows_copy(i, r):
            base = wid * EPW + i * CHUNK
            return pltpu.make_async_copy(rows_hbm.at[pl.ds(base, CHUNK)],
                                         sidx_b[r], rsem[r])

        def g_copy(b):
            return pltpu.make_async_copy(y_hbm.at[cidx_b[b]],
                                         g_b[b], gsem[b])

        def scat_copy(b, r):
            return pltpu.make_async_copy(o_b[b], acc_sh.at[sidx_b[r]],
                                         ssem[b])

        def compute(b):
            g_v = g_b[b]
            o_v = o_b[b]

            himask = jnp.full((16,), -65536, jnp.int32)

            @plsc.parallel_loop(0, CHUNK, unroll=8)
            def edge_body(e):
                wv = w_b[b][pl.ds(e * T, 16)]
                ws = (wv[0], wv[1], wv[2], wv[3])
                for k in range(F_OUT // 32):
                    acc_lo = jnp.zeros((16,), jnp.float32)
                    acc_hi = jnp.zeros((16,), jnp.float32)
                    for t in range(T):
                        v = g_v[e, pl.ds(t * (F_OUT // 2) + k * 16, 16)]
                        lo = plsc.bitcast(lax.shift_left(v, 16), jnp.float32)
                        hi = plsc.bitcast(v & himask, jnp.float32)
                        acc_lo = acc_lo + ws[t] * lo
                        acc_hi = acc_hi + ws[t] * hi
                    o_v[e, pl.ds(k * 32, 16)] = acc_lo
                    o_v[e, pl.ds(k * 32 + 16, 16)] = acc_hi

        def one_iter(i, r, b, wait_scat):
            g_copy(b).wait()
            if isinstance(i, int) and i + 1 >= NCHUNK:
                pass
            else:
                wait_idx(i + 1, 1 - b)
                g_copy(1 - b).start()
            rows_copy(i, r).wait()
            if wait_scat:
                scat_copy(b, (r + 2) % 4).wait()
            compute(b)
            pltpu.async_copy(o_b[b], acc_sh.at[sidx_b[r]], ssem[b],
                             add=True)
            if not isinstance(i, int) or i + 2 < NCHUNK:
                issue_idx(i + 2, b)
            if not isinstance(i, int) or i + 1 < NCHUNK:
                rows_copy(i + 1, (r + 1) % 4).start()

        issue_idx(0, 0)
        issue_idx(1, 1)
        rows_copy(0, 0).start()
        wait_idx(0, 0)
        g_copy(0).start()

        one_iter(0, 0, 0, False)
        one_iter(1, 1, 1, False)

        def quad_body(iq, _):
            for j in range(4):
                i = 2 + iq * 4 + j
                r = (2 + j) % 4
                one_iter(i, r, r % 2, True)
            return 0

        NQ = (NCHUNK - 2 - 4) // 4
        lax.fori_loop(0, NQ, quad_body, 0)
        for i in range(NCHUNK - 4, NCHUNK):
            one_iter(i, i % 4, i % 2, True)
        scat_copy(0, (NCHUNK - 2) % 4).wait()
        scat_copy(1, (NCHUNK - 1) % 4).wait()

        plsc.subcore_barrier()
        r0 = sid * ROWS_PER_SUB
        pltpu.sync_copy(acc_sh.at[pl.ds(r0, ROWS_PER_SUB), :],
                        out_hbm.at[cid, pl.ds(r0, ROWS_PER_SUB), :])

    return sc_kernel


def kernel(node_features, edge_features, indices, out_size, kernel, bias):
    kcat = jnp.transpose(kernel, (1, 0, 2)).reshape(F_IN, D)
    m = np.arange(D // 2)
    perm_a = (m // 64) * 128 + 32 * ((m % 64) // 16) + (m % 16)
    kcat2 = jnp.concatenate([kcat[:, perm_a], kcat[:, perm_a + 16]], axis=1)
    cols = indices[:, 1]
    rows = indices[:, 0]
    w_em = edge_features.T.reshape(-1)
    bias_adj = bias + (jnp.asarray(out_size, jnp.float32) - float(N))

    y_i32 = _matmul_y(node_features, kcat2)
    partials = _make_sc_kernel()(y_i32, cols, rows, w_em)
    return _epilogue(partials, bias_adj.reshape(1, F_OUT))


# speedup vs baseline: 1.6291x; 1.4845x over previous
import functools

import jax
import jax.numpy as jnp
import numpy as np
from jax import lax
from jax.experimental import pallas as pl
from jax.experimental.pallas import tpu as pltpu
from jax.experimental.pallas import tpu_sc as plsc

N = 10000
E = 320000
F_IN = 128
F_OUT = 128
T = 4
D = T * F_OUT

NC = 2
NS = 16
NW = NC * NS
EPW = E // NW
CHUNK = 40
NCHUNK = EPW // CHUNK
NPAD = 10240
ROWS_PER_SUB = NPAD // NS


def _matmul_y(x, kcat2):
    def body(x_ref, k_ref, o_ref):
        r = jnp.dot(x_ref[...], k_ref[...],
                    preferred_element_type=jnp.float32)
        ra = r[:, :D // 2].astype(jnp.bfloat16).astype(jnp.float32)
        rb = r[:, D // 2:].astype(jnp.bfloat16).astype(jnp.float32)
        abits = lax.shift_right_logical(
            lax.bitcast_convert_type(ra, jnp.int32), 16)
        bbits = lax.bitcast_convert_type(rb, jnp.int32) & jnp.int32(-65536)
        o_ref[...] = abits | bbits

    return pl.pallas_call(
        body,
        grid=(10,),
        in_specs=[
            pl.BlockSpec((N // 10, F_IN), lambda i: (i, 0)),
            pl.BlockSpec((F_IN, D), lambda i: (0, 0)),
        ],
        out_specs=pl.BlockSpec((N // 10, D // 2), lambda i: (i, 0)),
        out_shape=jax.ShapeDtypeStruct((N, D // 2), jnp.int32),
    )(x, kcat2)


def _epilogue(partials, bias_row):
    def body(p_ref, b_ref, o_ref):
        o_ref[...] = jnp.maximum(p_ref[0] + p_ref[1] + b_ref[...], 0.0)

    return pl.pallas_call(
        body,
        grid=(10,),
        in_specs=[
            pl.BlockSpec((2, N // 10, F_OUT), lambda i: (0, i, 0)),
            pl.BlockSpec((1, F_OUT), lambda i: (0, 0)),
        ],
        out_specs=pl.BlockSpec((N // 10, F_OUT), lambda i: (i, 0)),
        out_shape=jax.ShapeDtypeStruct((N, F_OUT), jnp.float32),
    )(partials, bias_row)


def _make_sc_kernel():
    mesh = plsc.VectorSubcoreMesh(core_axis_name="c", subcore_axis_name="s",
                                  num_cores=NC, num_subcores=NS)

    @functools.partial(
        pl.kernel,
        out_type=jax.ShapeDtypeStruct((NC, NPAD, F_OUT), jnp.float32),
        mesh=mesh,
        compiler_params=pltpu.CompilerParams(needs_layout_passes=False),
        scratch_types=[
            pltpu.VMEM((CHUNK,), jnp.int32),
            pltpu.VMEM((CHUNK,), jnp.int32),
            pltpu.VMEM((CHUNK,), jnp.int32),
            pltpu.VMEM((CHUNK,), jnp.int32),
            pltpu.VMEM((CHUNK,), jnp.int32),
            pltpu.VMEM((CHUNK,), jnp.int32),
            pltpu.VMEM((CHUNK + 16,), jnp.float32),
            pltpu.VMEM((CHUNK + 16,), jnp.float32),
            pltpu.VMEM((CHUNK + 16,), jnp.float32),
            pltpu.VMEM((CHUNK + 16,), jnp.float32),
            pltpu.VMEM((CHUNK + 16,), jnp.float32),
            pltpu.VMEM((CHUNK + 16,), jnp.float32),
            pltpu.VMEM((CHUNK + 16,), jnp.float32),
            pltpu.VMEM((CHUNK + 16,), jnp.float32),
            pltpu.VMEM((CHUNK, D // 2), jnp.int32),
            pltpu.VMEM((CHUNK, D // 2), jnp.int32),
            pltpu.VMEM((CHUNK, F_OUT), jnp.float32),
            pltpu.VMEM((CHUNK, F_OUT), jnp.float32),
            pltpu.VMEM_SHARED((NPAD, F_OUT), jnp.float32),
            pltpu.SemaphoreType.DMA,
            pltpu.SemaphoreType.DMA,
            pltpu.SemaphoreType.DMA,
            pltpu.SemaphoreType.DMA,
            pltpu.SemaphoreType.DMA,
            pltpu.SemaphoreType.DMA,
            pltpu.SemaphoreType.DMA,
            pltpu.SemaphoreType.DMA,
            pltpu.SemaphoreType.DMA,
            pltpu.SemaphoreType.DMA,
        ],
    )
    def sc_kernel(y_hbm, cols_hbm, rows_hbm, w_hbm, out_hbm,
                  cidx0_v, cidx1_v,
                  sidx0_v, sidx1_v, sidx2_v, sidx3_v,
                  w00_v, w01_v, w02_v, w03_v,
                  w10_v, w11_v, w12_v, w13_v,
                  g0_v, g1_v, o0_v, o1_v, acc_sh,
                  isem0, isem1, gsem0, gsem1,
                  rsem0, rsem1, rsem2, rsem3, ssem0, ssem1):
        cid = lax.axis_index("c")
        sid = lax.axis_index("s")
        cidx_b = (cidx0_v, cidx1_v)
        sidx_b = (sidx0_v, sidx1_v, sidx2_v, sidx3_v)
        w_b = ((w00_v, w01_v, w02_v, w03_v), (w10_v, w11_v, w12_v, w13_v))
        g_b = (g0_v, g1_v)
        o_b = (o0_v, o1_v)
        isem = (isem0, isem1)
        gsem = (gsem0, gsem1)
        rsem = (rsem0, rsem1, rsem2, rsem3)
        ssem = (ssem0, ssem1)

        zero16 = jnp.zeros((16,), jnp.float32)

        def zinit(i, _):
            for j in range(F_OUT // 16):
                o0_v[i, pl.ds(j * 16, 16)] = zero16
            return 0

        lax.fori_loop(0, CHUNK, zinit, 0)
        for zc in range(ROWS_PER_SUB // CHUNK):
            r0 = sid * ROWS_PER_SUB + zc * CHUNK
            pltpu.sync_copy(o0_v, acc_sh.at[pl.ds(r0, CHUNK), :])
        plsc.subcore_barrier()

        wid = sid * NC + cid

        def idx_copies(i, b):
            base = wid * EPW + i * CHUNK
            return (
                pltpu.make_async_copy(cols_hbm.at[pl.ds(base, CHUNK)],
                                      cidx_b[b], isem[b]),
            ) + tuple(
                pltpu.make_async_copy(
                    w_hbm.at[pl.ds(t * E + base, CHUNK)],
                    w_b[b][t].at[pl.ds(0, CHUNK)], isem[b])
                for t in range(T)
            )

        def issue_idx(i, b):
            for c in idx_copies(i, b):
                c.start()

        def wait_idx(i, b):
            for c in idx_copies(i, b):
                c.wait()

        def rows_copy(i, r):
            base = wid * EPW + i * CHUNK
            return pltpu.make_async_copy(rows_hbm.at[pl.ds(base, CHUNK)],
                                         sidx_b[r], rsem[r])

        def g_copy(b):
            return pltpu.make_async_copy(y_hbm.at[cidx_b[b]],
                                         g_b[b], gsem[b])

        def scat_copy(b, r):
            return pltpu.make_async_copy(o_b[b], acc_sh.at[sidx_b[r]],
                                         ssem[b])

        def compute(b):
            g_v = g_b[b]
            o_v = o_b[b]

            himask = jnp.full((16,), -65536, jnp.int32)

            @plsc.parallel_loop(0, CHUNK, unroll=8)
            def edge_body(e):
                ws = tuple(w_b[b][t][pl.ds(e, 16)][0] for t in range(T))
                for k in range(F_OUT // 32):
                    acc_lo = jnp.zeros((16,), jnp.float32)
                    acc_hi = jnp.zeros((16,), jnp.float32)
                    for t in range(T):
                        v = g_v[e, pl.ds(t * (F_OUT // 2) + k * 16, 16)]
                        lo = plsc.bitcast(lax.shift_left(v, 16), jnp.float32)
                        hi = plsc.bitcast(v & himask, jnp.float32)
                        acc_lo = acc_lo + ws[t] * lo
                        acc_hi = acc_hi + ws[t] * hi
                    o_v[e, pl.ds(k * 32, 16)] = acc_lo
                    o_v[e, pl.ds(k * 32 + 16, 16)] = acc_hi

        def one_iter(i, r, b, wait_scat):
            g_copy(b).wait()
            if isinstance(i, int) and i + 1 >= NCHUNK:
                pass
            else:
                wait_idx(i + 1, 1 - b)
                g_copy(1 - b).start()
            rows_copy(i, r).wait()
            if wait_scat:
                scat_copy(b, (r + 2) % 4).wait()
            compute(b)
            pltpu.async_copy(o_b[b], acc_sh.at[sidx_b[r]], ssem[b],
                             add=True)
            if not isinstance(i, int) or i + 2 < NCHUNK:
                issue_idx(i + 2, b)
            if not isinstance(i, int) or i + 1 < NCHUNK:
                rows_copy(i + 1, (r + 1) % 4).start()

        issue_idx(0, 0)
        issue_idx(1, 1)
        rows_copy(0, 0).start()
        wait_idx(0, 0)
        g_copy(0).start()

        one_iter(0, 0, 0, False)
        one_iter(1, 1, 1, False)

        def quad_body(iq, _):
            for j in range(4):
                i = 2 + iq * 4 + j
                r = (2 + j) % 4
                one_iter(i, r, r % 2, True)
            return 0

        NQ = (NCHUNK - 2 - 4) // 4
        lax.fori_loop(0, NQ, quad_body, 0)
        for i in range(NCHUNK - 4, NCHUNK):
            one_iter(i, i % 4, i % 2, True)
        scat_copy(0, (NCHUNK - 2) % 4).wait()
        scat_copy(1, (NCHUNK - 1) % 4).wait()

        plsc.subcore_barrier()
        r0 = sid * ROWS_PER_SUB
        pltpu.sync_copy(acc_sh.at[pl.ds(r0, ROWS_PER_SUB), :],
                        out_hbm.at[cid, pl.ds(r0, ROWS_PER_SUB), :])

    return sc_kernel


def kernel(node_features, edge_features, indices, out_size, kernel, bias):
    kcat = jnp.transpose(kernel, (1, 0, 2)).reshape(F_IN, D)
    m = np.arange(D // 2)
    perm_a = (m // 64) * 128 + 32 * ((m % 64) // 16) + (m % 16)
    kcat2 = jnp.concatenate([kcat[:, perm_a], kcat[:, perm_a + 16]], axis=1)
    cols = indices[:, 1]
    rows = indices[:, 0]
    w_flat = edge_features.reshape(-1)
    bias_adj = bias + (jnp.asarray(out_size, jnp.float32) - float(N))

    y_i32 = _matmul_y(node_features, kcat2)
    partials = _make_sc_kernel()(y_i32, cols, rows, w_flat)
    return _epilogue(partials, bias_adj.reshape(1, F_OUT))
